# Initial kernel scaffold; baseline (speedup 1.0000x reference)
import jax
import jax.numpy as jnp
from jax.experimental import pallas as pl


def kernel(z, edge_index, edge_weight, batch, W_rel0, b_rel0, W_root0, W_rel1, b_rel1, W_root1, W_rel2, b_rel2, W_root2, W_rel3, b_rel3, W_root3):
    raise NotImplementedError("write your pallas kernel here")


# speedup vs baseline: 2.8697x; 2.8697x over previous
import functools

import jax
import jax.numpy as jnp
from jax import lax
from jax.experimental import pallas as pl
from jax.experimental.pallas import tpu as pltpu
from jax.experimental.pallas import tpu_sc as plsc

N = 50000
E = 800000
F = 32
NW = 32
B = 128
E_PAD = 802816
EPW = E_PAD // NW
NCHUNK = EPW // B
N_PAD = 50176
RPT = N_PAD // 16
RCH = 112


def _seg_slab_kernel(nslab):
    mesh = plsc.VectorSubcoreMesh(core_axis_name="c", subcore_axis_name="s")

    @functools.partial(
        pl.kernel,
        mesh=mesh,
        compiler_params=pltpu.CompilerParams(use_tc_tiling_on_sc=False),
        out_type=jax.ShapeDtypeStruct((nslab, 2, N_PAD, F), jnp.float32),
        scratch_types=[
            pltpu.VMEM((B,), jnp.int32),
            pltpu.VMEM((B,), jnp.int32),
            pltpu.VMEM((B,), jnp.float32),
            pltpu.VMEM((B, F), jnp.float32),
            pltpu.VMEM((RCH, F), jnp.float32),
            pltpu.VMEM((RCH, F), jnp.float32),
            pltpu.VMEM_SHARED((N_PAD, F), jnp.float32),
            pltpu.SemaphoreType.DMA,
        ],
    )
    def k(*refs):
        x_refs = refs[:nslab]
        src_hbm, dst_hbm, w_hbm, out_hbm = refs[nslab:nslab + 4]
        src_v, dst_v, w_v, rows_v, zbuf_v, stage_v, spacc, sem = refs[nslab + 4:]

        c = lax.axis_index("c")
        s = lax.axis_index("s")
        wid = s * 2 + c
        zeros16 = jnp.zeros((16,), jnp.float32)

        def zero_stage(r, _):
            for j in range(F // 16):
                zbuf_v[r, pl.ds(j * 16, 16)] = zeros16
            return 0

        lax.fori_loop(0, RCH, zero_stage, 0)

        for slab in range(nslab):
            x_hbm = x_refs[slab]

            def zero_acc(kk, _):
                pltpu.sync_copy(zbuf_v,
                                spacc.at[pl.ds(s * RPT + kk * RCH, RCH)])
                return 0

            lax.fori_loop(0, RPT // RCH, zero_acc, 0)
            plsc.subcore_barrier()

            def do_chunk(ch, _):
                base = wid * EPW + ch * B
                pltpu.sync_copy(src_hbm.at[pl.ds(base, B)], src_v)
                pltpu.sync_copy(dst_hbm.at[pl.ds(base, B)], dst_v)
                pltpu.sync_copy(w_hbm.at[pl.ds(base, B)], w_v)
                pltpu.async_copy(x_hbm.at[src_v], rows_v, sem).wait()

                def scale(g, _):
                    w16 = w_v[pl.ds(g * 16, 16)]
                    for lane in range(16):
                        e = g * 16 + lane
                        wv = w16[lane]
                        for j in range(F // 16):
                            sl = pl.ds(j * 16, 16)
                            rows_v[e, sl] = rows_v[e, sl] * wv
                    return 0

                lax.fori_loop(0, B // 16, scale, 0)
                pltpu.sync_copy(rows_v, spacc.at[dst_v], add=True)
                return 0

            lax.fori_loop(0, NCHUNK, do_chunk, 0)
            plsc.subcore_barrier()

            def writeout(kk, _):
                r0 = s * RPT + kk * RCH
                pltpu.sync_copy(spacc.at[pl.ds(r0, RCH)], stage_v)
                pltpu.sync_copy(stage_v,
                                out_hbm.at[slab, c, pl.ds(r0, RCH)])
                return 0

            lax.fori_loop(0, RPT // RCH, writeout, 0)
            plsc.subcore_barrier()

    return k


_seg1 = _seg_slab_kernel(1)
_seg2 = _seg_slab_kernel(2)


def _dense_block(parts, x, Wr_splits, br, Wt, skip):
    d = x.shape[1]
    nsl = len(Wr_splits)
    ROWS = 400
    grid = (N // ROWS,)

    def body(*refs):
        part_refs = refs[:2 * nsl]
        x_ref, br_ref = refs[2 * nsl], refs[2 * nsl + 1]
        w_refs = refs[2 * nsl + 2:2 * nsl + 2 + nsl]
        wt_ref, o_ref = refs[-2], refs[-1]
        acc = br_ref[...] + jnp.dot(x_ref[...], wt_ref[...],
                                    preferred_element_type=jnp.float32)
        for i in range(nsl):
            agg = part_refs[2 * i][...] + part_refs[2 * i + 1][...]
            acc = acc + jnp.dot(agg, w_refs[i][...],
                                preferred_element_type=jnp.float32)
        acc = jnp.maximum(acc, 0.0)
        if skip:
            acc = jnp.maximum(acc + x_ref[...], 0.0)
        o_ref[...] = acc

    row_spec = lambda w: pl.BlockSpec((ROWS, w), lambda i: (i, 0))
    full_spec = lambda a, b: pl.BlockSpec((a, b), lambda i: (0, 0))
    in_specs = (
        [row_spec(F)] * (2 * nsl)
        + [row_spec(d), full_spec(1, 64)]
        + [full_spec(F, 64)] * nsl
        + [full_spec(d, 64)]
    )
    return pl.pallas_call(
        body,
        grid=grid,
        in_specs=in_specs,
        out_specs=row_spec(64),
        out_shape=jax.ShapeDtypeStruct((N, 64), jnp.float32),
    )(*parts, x, br.reshape(1, 64), *Wr_splits, Wt)


def kernel(z, edge_index, edge_weight, batch,
           W_rel0, b_rel0, W_root0,
           W_rel1, b_rel1, W_root1,
           W_rel2, b_rel2, W_root2,
           W_rel3, b_rel3, W_root3):
    pad = E_PAD - E
    src = jnp.concatenate([edge_index[0], jnp.zeros((pad,), jnp.int32)])
    dst = jnp.concatenate([edge_index[1], jnp.zeros((pad,), jnp.int32)])
    w = jnp.concatenate([edge_weight, jnp.zeros((pad,), jnp.float32)])

    def block(x, Wr, br, Wt, skip):
        d = x.shape[1]
        if d == F:
            p = _seg1(x, src, dst, w)
            parts = (p[0, 0, :N], p[0, 1, :N])
            Wr_splits = (Wr,)
        else:
            xa = x[:, :F]
            xb = x[:, F:]
            p = _seg2(xa, xb, src, dst, w)
            parts = (p[0, 0, :N], p[0, 1, :N], p[1, 0, :N], p[1, 1, :N])
            Wr_splits = (Wr[:F], Wr[F:])
        return _dense_block(parts, x, Wr_splits, br, Wt, skip)

    h = block(z, W_rel0, b_rel0, W_root0, False)
    h = block(h, W_rel1, b_rel1, W_root1, True)
    h = block(h, W_rel2, b_rel2, W_root2, True)
    h = block(h, W_rel3, b_rel3, W_root3, False)
    return h


# speedup vs baseline: 4.6841x; 1.6323x over previous
import functools

import jax
import jax.numpy as jnp
from jax import lax
from jax.experimental import pallas as pl
from jax.experimental.pallas import tpu as pltpu
from jax.experimental.pallas import tpu_sc as plsc

N = 50000
E = 800000
F = 32
NW = 32
B = 64
NCHUNK = 396
EPW = NCHUNK * B
E_PAD = EPW * NW
NBUF = 6
N_PAD = 50176
RPT = N_PAD // 16
RCH = 112


def _seg_slab_kernel(nslab):
    mesh = plsc.VectorSubcoreMesh(core_axis_name="c", subcore_axis_name="s")

    scratch = (
        [pltpu.VMEM((B,), jnp.int32) for _ in range(NBUF)]
        + [pltpu.VMEM((B,), jnp.float32) for _ in range(NBUF)]
        + [pltpu.VMEM((B,), jnp.int32) for _ in range(NBUF)]
        + [pltpu.VMEM((B, F), jnp.float32) for _ in range(NBUF)]
        + [
            pltpu.VMEM((RCH, F), jnp.float32),
            pltpu.VMEM((RCH, F), jnp.float32),
            pltpu.VMEM_SHARED((N_PAD, F), jnp.float32),
        ]
        + [pltpu.SemaphoreType.DMA for _ in range(4 * NBUF)]
    )

    @functools.partial(
        pl.kernel,
        mesh=mesh,
        compiler_params=pltpu.CompilerParams(use_tc_tiling_on_sc=False),
        out_type=jax.ShapeDtypeStruct((nslab, 2, N_PAD, F), jnp.float32),
        scratch_types=scratch,
    )
    def k(*refs):
        x_refs = refs[:nslab]
        src_hbm, dst_hbm, w_hbm, out_hbm = refs[nslab:nslab + 4]
        rest = refs[nslab + 4:]
        src_v = rest[0:NBUF]
        w_v = rest[NBUF:2 * NBUF]
        dst_v = rest[2 * NBUF:3 * NBUF]
        rows_v = rest[3 * NBUF:4 * NBUF]
        zbuf_v, stage_v, spacc = rest[4 * NBUF:4 * NBUF + 3]
        sems = rest[4 * NBUF + 3:]
        se = sems[0:NBUF]
        sd = sems[NBUF:2 * NBUF]
        sg = sems[2 * NBUF:3 * NBUF]
        ss = sems[3 * NBUF:4 * NBUF]

        c = lax.axis_index("c")
        s = lax.axis_index("s")
        wid = s * 2 + c
        ebase = wid * EPW
        zeros16 = jnp.zeros((16,), jnp.float32)

        def zero_stage(r, _):
            for j in range(F // 16):
                zbuf_v[r, pl.ds(j * 16, 16)] = zeros16
            return 0

        lax.fori_loop(0, RCH, zero_stage, 0)

        def issue_ew(ch, b):
            base = ebase + ch * B
            pltpu.async_copy(src_hbm.at[pl.ds(base, B)], src_v[b], se[b])
            pltpu.async_copy(w_hbm.at[pl.ds(base, B)], w_v[b], se[b])

        def wait_ew(b):
            pltpu.make_async_copy(src_hbm.at[pl.ds(0, B)], src_v[b],
                                  se[b]).wait()
            pltpu.make_async_copy(w_hbm.at[pl.ds(0, B)], w_v[b],
                                  se[b]).wait()

        def issue_dst(ch, b):
            base = ebase + ch * B
            pltpu.async_copy(dst_hbm.at[pl.ds(base, B)], dst_v[b], sd[b])

        def wait_dst(b):
            pltpu.make_async_copy(dst_hbm.at[pl.ds(0, B)], dst_v[b],
                                  sd[b]).wait()

        for slab in range(nslab):
            x_hbm = x_refs[slab]

            def scale(b):
                def scale_grp(g, _):
                    w16 = w_v[b][pl.ds(g * 16, 16)]
                    for lane in range(16):
                        e = g * 16 + lane
                        wv = w16[lane]
                        for j in range(F // 16):
                            sl = pl.ds(j * 16, 16)
                            rows_v[b][e, sl] = rows_v[b][e, sl] * wv
                    return 0

                lax.fori_loop(0, B // 16, scale_grp, 0)

            def zero_acc(kk, _):
                pltpu.sync_copy(zbuf_v,
                                spacc.at[pl.ds(s * RPT + kk * RCH, RCH)])
                return 0

            lax.fori_loop(0, RPT // RCH, zero_acc, 0)
            plsc.subcore_barrier()

            for b in range(NBUF):
                issue_ew(b, b)
            issue_dst(0, 0)
            wait_ew(0)
            pltpu.async_copy(x_hbm.at[src_v[0]], rows_v[0], sg[0])

            def outer(kk, _):
                for u in range(NBUF):
                    bcur = u
                    bnext = (u + 1) % NBUF
                    ch = NBUF * kk + u

                    def w1():
                        pltpu.make_async_copy(
                            rows_v[bnext],
                            spacc.at[dst_v[bnext]], ss[bnext]).wait()

                    if u == 5:
                        w1()
                    else:
                        pl.when(kk > 0)(w1)

                    def advance():
                        issue_dst(ch + 1, bnext)
                        wait_ew(bnext)
                        pltpu.async_copy(x_hbm.at[src_v[bnext]],
                                         rows_v[bnext], sg[bnext])

                    if u == 5:
                        pl.when(kk < (NCHUNK // NBUF) - 1)(advance)
                    else:
                        advance()

                    pltpu.make_async_copy(x_hbm.at[src_v[bcur]],
                                          rows_v[bcur], sg[bcur]).wait()
                    scale(bcur)

                    def reload():
                        issue_ew(ch + NBUF, bcur)

                    pl.when(kk < (NCHUNK // NBUF) - 1)(reload)

                    wait_dst(bcur)
                    pltpu.async_copy(rows_v[bcur], spacc.at[dst_v[bcur]],
                                     ss[bcur], add=True)
                return 0

            lax.fori_loop(0, NCHUNK // NBUF, outer, 0)

            for i in range(5):
                b = (NCHUNK - 5 + i) % NBUF
                pltpu.make_async_copy(rows_v[b], spacc.at[dst_v[b]],
                                      ss[b]).wait()
            plsc.subcore_barrier()

            def writeout(kk, _):
                r0 = s * RPT + kk * RCH
                pltpu.sync_copy(spacc.at[pl.ds(r0, RCH)], stage_v)
                pltpu.sync_copy(stage_v,
                                out_hbm.at[slab, c, pl.ds(r0, RCH)])
                return 0

            lax.fori_loop(0, RPT // RCH, writeout, 0)
            plsc.subcore_barrier()

    return k


_seg1 = _seg_slab_kernel(1)
_seg2 = _seg_slab_kernel(2)


def _dense_block(parts, x, Wr_splits, br, Wt, skip):
    d = x.shape[1]
    nsl = len(Wr_splits)
    ROWS = 400
    grid = (N // ROWS,)

    def body(*refs):
        part_refs = refs[:2 * nsl]
        x_ref, br_ref = refs[2 * nsl], refs[2 * nsl + 1]
        w_refs = refs[2 * nsl + 2:2 * nsl + 2 + nsl]
        wt_ref, o_ref = refs[-2], refs[-1]
        acc = br_ref[...] + jnp.dot(x_ref[...], wt_ref[...],
                                    preferred_element_type=jnp.float32)
        for i in range(nsl):
            agg = part_refs[2 * i][...] + part_refs[2 * i + 1][...]
            acc = acc + jnp.dot(agg, w_refs[i][...],
                                preferred_element_type=jnp.float32)
        acc = jnp.maximum(acc, 0.0)
        if skip:
            acc = jnp.maximum(acc + x_ref[...], 0.0)
        o_ref[...] = acc

    row_spec = lambda w: pl.BlockSpec((ROWS, w), lambda i: (i, 0))
    full_spec = lambda a, b: pl.BlockSpec((a, b), lambda i: (0, 0))
    in_specs = (
        [row_spec(F)] * (2 * nsl)
        + [row_spec(d), full_spec(1, 64)]
        + [full_spec(F, 64)] * nsl
        + [full_spec(d, 64)]
    )
    return pl.pallas_call(
        body,
        grid=grid,
        in_specs=in_specs,
        out_specs=row_spec(64),
        out_shape=jax.ShapeDtypeStruct((N, 64), jnp.float32),
    )(*parts, x, br.reshape(1, 64), *Wr_splits, Wt)


def kernel(z, edge_index, edge_weight, batch,
           W_rel0, b_rel0, W_root0,
           W_rel1, b_rel1, W_root1,
           W_rel2, b_rel2, W_root2,
           W_rel3, b_rel3, W_root3):
    pad = E_PAD - E
    src = jnp.concatenate([edge_index[0], jnp.zeros((pad,), jnp.int32)])
    dst = jnp.concatenate([edge_index[1], jnp.zeros((pad,), jnp.int32)])
    w = jnp.concatenate([edge_weight, jnp.zeros((pad,), jnp.float32)])

    def block(x, Wr, br, Wt, skip):
        d = x.shape[1]
        if d == F:
            p = _seg1(x, src, dst, w)
            parts = (p[0, 0, :N], p[0, 1, :N])
            Wr_splits = (Wr,)
        else:
            xa = x[:, :F]
            xb = x[:, F:]
            p = _seg2(xa, xb, src, dst, w)
            parts = (p[0, 0, :N], p[0, 1, :N], p[1, 0, :N], p[1, 1, :N])
            Wr_splits = (Wr[:F], Wr[F:])
        return _dense_block(parts, x, Wr_splits, br, Wt, skip)

    h = block(z, W_rel0, b_rel0, W_root0, False)
    h = block(h, W_rel1, b_rel1, W_root1, True)
    h = block(h, W_rel2, b_rel2, W_root2, True)
    h = block(h, W_rel3, b_rel3, W_root3, False)
    return h


# speedup vs baseline: 5.1075x; 1.0904x over previous
import functools

import jax
import jax.numpy as jnp
from jax import lax
from jax.experimental import pallas as pl
from jax.experimental.pallas import tpu as pltpu
from jax.experimental.pallas import tpu_sc as plsc

N = 50000
E = 800000
F = 32
NW = 32
B = 64
NCHUNK = 396
EPW = NCHUNK * B
E_PAD = EPW * NW
NBUF = 6
N_PAD = 50176
RPT = N_PAD // 16
RCH = 112


def _seg_slab_kernel(nslab):
    mesh = plsc.VectorSubcoreMesh(core_axis_name="c", subcore_axis_name="s")

    scratch = (
        [pltpu.VMEM((B,), jnp.int32) for _ in range(NBUF)]
        + [pltpu.VMEM((B,), jnp.float32) for _ in range(NBUF)]
        + [pltpu.VMEM((B,), jnp.int32) for _ in range(NBUF)]
        + [pltpu.VMEM((B, F), jnp.float32) for _ in range(NBUF)]
        + [
            pltpu.VMEM((RCH, F), jnp.float32),
            pltpu.VMEM((RCH, F), jnp.float32),
            pltpu.VMEM_SHARED((N_PAD, F), jnp.float32),
        ]
        + [pltpu.SemaphoreType.DMA for _ in range(4 * NBUF)]
    )

    @functools.partial(
        pl.kernel,
        mesh=mesh,
        compiler_params=pltpu.CompilerParams(use_tc_tiling_on_sc=False),
        out_type=jax.ShapeDtypeStruct((nslab, 2, N_PAD, F), jnp.float32),
        scratch_types=scratch,
    )
    def k(*refs):
        x_refs = refs[:nslab]
        src_hbm, dst_hbm, w_hbm, out_hbm = refs[nslab:nslab + 4]
        rest = refs[nslab + 4:]
        src_v = rest[0:NBUF]
        w_v = rest[NBUF:2 * NBUF]
        dst_v = rest[2 * NBUF:3 * NBUF]
        rows_v = rest[3 * NBUF:4 * NBUF]
        zbuf_v, stage_v, spacc = rest[4 * NBUF:4 * NBUF + 3]
        sems = rest[4 * NBUF + 3:]
        se = sems[0:NBUF]
        sd = sems[NBUF:2 * NBUF]
        sg = sems[2 * NBUF:3 * NBUF]
        ss = sems[3 * NBUF:4 * NBUF]

        c = lax.axis_index("c")
        s = lax.axis_index("s")
        wid = s * 2 + c
        ebase = wid * EPW
        zeros16 = jnp.zeros((16,), jnp.float32)

        def zero_stage(r, _):
            for j in range(F // 16):
                zbuf_v[r, pl.ds(j * 16, 16)] = zeros16
            return 0

        lax.fori_loop(0, RCH, zero_stage, 0)

        def issue_ew(ch, b):
            base = ebase + ch * B
            pltpu.async_copy(src_hbm.at[pl.ds(base, B)], src_v[b], se[b])
            pltpu.async_copy(w_hbm.at[pl.ds(base, B)], w_v[b], se[b])

        def wait_ew(b):
            pltpu.make_async_copy(src_hbm.at[pl.ds(0, B)], src_v[b],
                                  se[b]).wait()
            pltpu.make_async_copy(w_hbm.at[pl.ds(0, B)], w_v[b],
                                  se[b]).wait()

        def issue_dst(ch, b):
            base = ebase + ch * B
            pltpu.async_copy(dst_hbm.at[pl.ds(base, B)], dst_v[b], sd[b])

        def wait_dst(b):
            pltpu.make_async_copy(dst_hbm.at[pl.ds(0, B)], dst_v[b],
                                  sd[b]).wait()

        for slab in range(nslab):
            x_hbm = x_refs[slab]

            def scale(b):
                def scale_grp(g, _):
                    w16 = w_v[b][pl.ds(g * 16, 16)]
                    for lane in range(16):
                        e = g * 16 + lane
                        wv = w16[lane]
                        for j in range(F // 16):
                            sl = pl.ds(j * 16, 16)
                            rows_v[b][e, sl] = rows_v[b][e, sl] * wv
                    return 0

                lax.fori_loop(0, B // 16, scale_grp, 0)

            def zero_acc(kk, _):
                pltpu.sync_copy(zbuf_v,
                                spacc.at[pl.ds(s * RPT + kk * RCH, RCH)])
                return 0

            lax.fori_loop(0, RPT // RCH, zero_acc, 0)
            plsc.subcore_barrier()

            for b in range(NBUF):
                issue_ew(b, b)
            issue_dst(0, 0)
            wait_ew(0)
            pltpu.async_copy(x_hbm.at[src_v[0]], rows_v[0], sg[0])

            def outer(kk, _):
                for u in range(NBUF):
                    bcur = u
                    bnext = (u + 1) % NBUF
                    ch = NBUF * kk + u

                    def w1():
                        pltpu.make_async_copy(
                            rows_v[bnext],
                            spacc.at[dst_v[bnext]], ss[bnext]).wait()

                    if u == 5:
                        w1()
                    else:
                        pl.when(kk > 0)(w1)

                    def advance():
                        issue_dst(ch + 1, bnext)
                        wait_ew(bnext)
                        pltpu.async_copy(x_hbm.at[src_v[bnext]],
                                         rows_v[bnext], sg[bnext])

                    if u == 5:
                        pl.when(kk < (NCHUNK // NBUF) - 1)(advance)
                    else:
                        advance()

                    pltpu.make_async_copy(x_hbm.at[src_v[bcur]],
                                          rows_v[bcur], sg[bcur]).wait()
                    scale(bcur)

                    def reload():
                        issue_ew(ch + NBUF, bcur)

                    pl.when(kk < (NCHUNK // NBUF) - 1)(reload)

                    wait_dst(bcur)
                    pltpu.async_copy(rows_v[bcur], spacc.at[dst_v[bcur]],
                                     ss[bcur], add=True)
                return 0

            lax.fori_loop(0, NCHUNK // NBUF, outer, 0)

            for i in range(5):
                b = (NCHUNK - 5 + i) % NBUF
                pltpu.make_async_copy(rows_v[b], spacc.at[dst_v[b]],
                                      ss[b]).wait()
            plsc.subcore_barrier()

            def writeout(kk, _):
                r0 = s * RPT + kk * RCH
                pltpu.sync_copy(spacc.at[pl.ds(r0, RCH)], stage_v)
                pltpu.sync_copy(stage_v,
                                out_hbm.at[slab, c, pl.ds(r0, RCH)])
                return 0

            lax.fori_loop(0, RPT // RCH, writeout, 0)
            plsc.subcore_barrier()

    return k


_seg1 = _seg_slab_kernel(1)
_seg2 = _seg_slab_kernel(2)


def _dense_block(p, x, Wr_splits, br, Wt, skip):
    d = x.shape[1]
    nsl = len(Wr_splits)
    ROWS = 400
    grid = (N // ROWS,)

    def body(*refs):
        part_refs = refs[:2 * nsl]
        x_ref, br_ref = refs[2 * nsl], refs[2 * nsl + 1]
        w_refs = refs[2 * nsl + 2:2 * nsl + 2 + nsl]
        wt_ref, o_ref = refs[-2], refs[-1]
        acc = br_ref[...] + jnp.dot(x_ref[...], wt_ref[...],
                                    preferred_element_type=jnp.float32)
        for i in range(nsl):
            agg = part_refs[2 * i][0, 0] + part_refs[2 * i + 1][0, 0]
            acc = acc + jnp.dot(agg, w_refs[i][...],
                                preferred_element_type=jnp.float32)
        acc = jnp.maximum(acc, 0.0)
        if skip:
            acc = jnp.maximum(acc + x_ref[...], 0.0)
        o_ref[...] = acc

    row_spec = lambda w: pl.BlockSpec((ROWS, w), lambda i: (i, 0))
    full_spec = lambda a, b: pl.BlockSpec((a, b), lambda i: (0, 0))
    part_specs = [
        pl.BlockSpec((1, 1, ROWS, F),
                     (lambda i, sl=sl, cc=cc: (sl, cc, i, 0)))
        for sl in range(nsl) for cc in range(2)
    ]
    in_specs = (
        part_specs
        + [row_spec(d), full_spec(1, 64)]
        + [full_spec(F, 64)] * nsl
        + [full_spec(d, 64)]
    )
    return pl.pallas_call(
        body,
        grid=grid,
        in_specs=in_specs,
        out_specs=row_spec(64),
        out_shape=jax.ShapeDtypeStruct((N, 64), jnp.float32),
    )(*([p] * (2 * nsl)), x, br.reshape(1, 64), *Wr_splits, Wt)


def kernel(z, edge_index, edge_weight, batch,
           W_rel0, b_rel0, W_root0,
           W_rel1, b_rel1, W_root1,
           W_rel2, b_rel2, W_root2,
           W_rel3, b_rel3, W_root3):
    pad = E_PAD - E
    src = jnp.concatenate([edge_index[0], jnp.zeros((pad,), jnp.int32)])
    dst = jnp.concatenate([edge_index[1], jnp.zeros((pad,), jnp.int32)])
    w = jnp.concatenate([edge_weight, jnp.zeros((pad,), jnp.float32)])

    def block(x, Wr, br, Wt, skip):
        d = x.shape[1]
        if d == F:
            p = _seg1(x, src, dst, w)
            Wr_splits = (Wr,)
        else:
            xa = x[:, :F]
            xb = x[:, F:]
            p = _seg2(xa, xb, src, dst, w)
            Wr_splits = (Wr[:F], Wr[F:])
        return _dense_block(p, x, Wr_splits, br, Wt, skip)

    h = block(z, W_rel0, b_rel0, W_root0, False)
    h = block(h, W_rel1, b_rel1, W_root1, True)
    h = block(h, W_rel2, b_rel2, W_root2, True)
    h = block(h, W_rel3, b_rel3, W_root3, False)
    return h
